# baseline (speedup 1.0000x reference)
import functools

import jax
import jax.numpy as jnp
from jax import lax
from jax.experimental import pallas as pl
from jax.experimental.pallas import tpu as pltpu


def _loss_body(grid, B, D, lvl_ref, batch_ref, ew_ref, bw_ref,
               logits_ref, data_ref, out_ref, logits_out_ref, s_ref, c_ref):
    step = pl.program_id(0)

    @pl.when(step == 0)
    def _init():
        s_ref[...] = jnp.zeros_like(s_ref)
        c_ref[...] = jnp.zeros_like(c_ref)

    lg = logits_ref[...]
    logits_out_ref[...] = lg
    diff = lg - data_ref[...]
    row = jnp.sum(diff * diff, axis=1) * (1.0 / D)
    wl = row * ew_ref[...]
    ids = batch_ref[...]
    rows = ids.shape[0]
    iot = lax.broadcasted_iota(jnp.int32, (B, rows), 0)
    mask = iot == ids[None, :]
    s_ref[0, :] += jnp.sum(jnp.where(mask, wl[None, :], 0.0), axis=1)
    c_ref[0, :] += jnp.sum(mask.astype(jnp.float32), axis=1)

    @pl.when(step == grid - 1)
    def _fin():
        seg = s_ref[0, :] / jnp.clip(c_ref[0, :], 1.0, None)
        seg = seg * bw_ref[...]
        lvl = lvl_ref[0]
        seg = jnp.clip(seg, 0.0, lvl)
        out_ref[0] = jnp.sum(seg) * (1.0 / B)


def kernel(batch, logits, data, batch_weight, element_weight, level):
    N, D = logits.shape
    B = batch_weight.shape[0]
    ROWS = 2048
    grid = N // ROWS
    lvl = jnp.asarray(level, jnp.float32).reshape(1)

    body = functools.partial(_loss_body, grid, B, D)

    loss, logits_out = pl.pallas_call(
        body,
        grid=(grid,),
        in_specs=[
            pl.BlockSpec(memory_space=pltpu.MemorySpace.SMEM),
            pl.BlockSpec((ROWS,), lambda i: (i,)),
            pl.BlockSpec((ROWS,), lambda i: (i,)),
            pl.BlockSpec((B,), lambda i: (0,)),
            pl.BlockSpec((ROWS, D), lambda i: (i, 0)),
            pl.BlockSpec((ROWS, D), lambda i: (i, 0)),
        ],
        out_specs=[
            pl.BlockSpec(memory_space=pltpu.MemorySpace.SMEM),
            pl.BlockSpec((ROWS, D), lambda i: (i, 0)),
        ],
        out_shape=[
            jax.ShapeDtypeStruct((1,), jnp.float32),
            jax.ShapeDtypeStruct((N, D), jnp.float32),
        ],
        scratch_shapes=[
            pltpu.VMEM((1, B), jnp.float32),
            pltpu.VMEM((1, B), jnp.float32),
        ],
    )(lvl, batch, element_weight, batch_weight, logits, data)
    return (loss[0], logits_out)


# speedup vs baseline: 1.1345x; 1.1345x over previous
import functools

import jax
import jax.numpy as jnp
from jax import lax
from jax.experimental import pallas as pl
from jax.experimental.pallas import tpu as pltpu


def _loss_body(grid, B, D, lvl_ref, batch_ref, ew_ref, bw_ref,
               logits_ref, data_ref, out_ref, logits_out_ref, s_ref, c_ref):
    step = pl.program_id(0)

    @pl.when(step == 0)
    def _init():
        s_ref[...] = jnp.zeros_like(s_ref)
        c_ref[...] = jnp.zeros_like(c_ref)

    lg = logits_ref[...]
    logits_out_ref[...] = lg
    diff = lg - data_ref[...]
    row = jnp.sum(diff * diff, axis=1) * (1.0 / D)
    wl = row * ew_ref[...]
    ids = batch_ref[...]
    rows = ids.shape[0]
    iot = lax.broadcasted_iota(jnp.int32, (B, rows), 0)
    mask = iot == ids[None, :]
    s_ref[0, :] += jnp.sum(jnp.where(mask, wl[None, :], 0.0), axis=1)
    c_ref[0, :] += jnp.sum(mask.astype(jnp.float32), axis=1)

    @pl.when(step == grid - 1)
    def _fin():
        seg = s_ref[0, :] / jnp.clip(c_ref[0, :], 1.0, None)
        seg = seg * bw_ref[...]
        lvl = lvl_ref[0]
        seg = jnp.clip(seg, 0.0, lvl)
        out_ref[0] = jnp.sum(seg) * (1.0 / B)


def kernel(batch, logits, data, batch_weight, element_weight, level):
    N, D = logits.shape
    B = batch_weight.shape[0]
    ROWS = 8192
    grid = N // ROWS
    lvl = jnp.asarray(level, jnp.float32).reshape(1)

    body = functools.partial(_loss_body, grid, B, D)

    loss, logits_out = pl.pallas_call(
        body,
        grid=(grid,),
        in_specs=[
            pl.BlockSpec(memory_space=pltpu.MemorySpace.SMEM),
            pl.BlockSpec((ROWS,), lambda i: (i,)),
            pl.BlockSpec((ROWS,), lambda i: (i,)),
            pl.BlockSpec((B,), lambda i: (0,)),
            pl.BlockSpec((ROWS, D), lambda i: (i, 0)),
            pl.BlockSpec((ROWS, D), lambda i: (i, 0)),
        ],
        out_specs=[
            pl.BlockSpec(memory_space=pltpu.MemorySpace.SMEM),
            pl.BlockSpec((ROWS, D), lambda i: (i, 0)),
        ],
        out_shape=[
            jax.ShapeDtypeStruct((1,), jnp.float32),
            jax.ShapeDtypeStruct((N, D), jnp.float32),
        ],
        scratch_shapes=[
            pltpu.VMEM((1, B), jnp.float32),
            pltpu.VMEM((1, B), jnp.float32),
        ],
    )(lvl, batch, element_weight, batch_weight, logits, data)
    return (loss[0], logits_out)
